# speedup vs baseline: 5.1597x; 5.1597x over previous
import functools

import jax
import jax.numpy as jnp
from jax import lax
from jax.experimental import pallas as pl
from jax.experimental.pallas import tpu as pltpu
from jax.experimental.pallas import tpu_sc as plsc

A_ = 256
B_ = 16
M_ = 16
K_ = 3
T_ = 128
NW_ = 32
CH_ = 128


@functools.lru_cache(maxsize=None)
def _make_gather(n_pad: int):
    edges = n_pad * M_
    per_w = edges // NW_
    n_ch = per_w // CH_
    mesh = plsc.VectorSubcoreMesh(core_axis_name="c", subcore_axis_name="s")

    @functools.partial(
        pl.kernel,
        out_type=jax.ShapeDtypeStruct((edges, A_), jnp.float32),
        mesh=mesh,
        scratch_types=[
            pltpu.VMEM((per_w,), jnp.int32),
            pltpu.VMEM((CH_, A_), jnp.float32),
            pltpu.SemaphoreType.DMA,
        ],
    )
    def gather_k(table_hbm, idx_hbm, out_hbm, idx_v, rows_v, sem):
        wid = lax.axis_index("s") * 2 + lax.axis_index("c")
        base = wid * per_w
        pltpu.sync_copy(idx_hbm.at[pl.ds(base, per_w)], idx_v)

        def body(i, carry):
            off = i * CH_
            pltpu.async_copy(
                table_hbm.at[idx_v.at[pl.ds(off, CH_)]], rows_v, sem
            ).wait()
            pltpu.sync_copy(rows_v, out_hbm.at[pl.ds(base + off, CH_)])
            return carry

        lax.fori_loop(0, n_ch, body, 0)

    return gather_k


def _dot(a, b):
    return lax.dot_general(
        a, b, (((1,), (0,)), ((), ())), preferred_element_type=jnp.float32
    )


def _tc_body(g_ref, e_ref, at_ref,
             wnF, wnC, wnE, weF, weC, weE, wsF, wsC, wsE,
             bF, bC, bE, s2, t2,
             afw, afb, nfw, nfb,
             out_ref, nn_ref):
    G = g_ref[...]
    E = e_ref[...]
    At = at_ref[...]

    KA = K_ * A_
    KB = K_ * B_

    XF = _dot(G, wnF[...]) + _dot(E, weF[...])
    XC = _dot(G, wnC[...]) + _dot(E, weC[...])
    XE = _dot(G, wnE[...]) + _dot(E, weE[...])
    SF = _dot(At, wsF[...]) + bF[0][None, :]
    SC = _dot(At, wsC[...]) + bC[0][None, :]
    SE = _dot(At, wsE[...]) + bE[0][None, :]

    F = XF.reshape(T_, M_, KA) + SF[:, None, :]
    C = XC.reshape(T_, M_, KA) + SC[:, None, :]

    mx = jnp.max(F, axis=1, keepdims=True)
    ef = jnp.exp(F - mx)
    se = jnp.sum(ef, axis=1)
    Cr = jnp.maximum(C, 0.0)
    ns = jnp.sum(ef * Cr, axis=1) / se
    ns = ns * s2[0][None, :] + t2[0][None, :]

    O = [At + ns[:, k * A_:(k + 1) * A_] for k in range(K_)]
    Gj = [afw[j, 0] * O[0] + afw[j, 1] * O[1] + afw[j, 2] * O[2] + afb[j]
          for j in range(2 * K_)]
    m2 = jnp.maximum(jnp.maximum(Gj[3], Gj[4]), Gj[5])
    e3 = jnp.exp(Gj[3] - m2)
    e4 = jnp.exp(Gj[4] - m2)
    e5 = jnp.exp(Gj[5] - m2)
    out_ref[...] = (Gj[0] * e3 + Gj[1] * e4 + Gj[2] * e5) / (e3 + e4 + e5)

    gE = XE.reshape(T_, M_, KB) + SE[:, None, :]
    nbr3 = E.reshape(T_, M_, B_)
    V = [gE[:, :, k * B_:(k + 1) * B_] + nbr3 for k in range(K_)]
    Nj = [nfw[j, 0] * V[0] + nfw[j, 1] * V[1] + nfw[j, 2] * V[2] + nfb[j]
          for j in range(2 * K_)]
    m3 = jnp.maximum(jnp.maximum(Nj[3], Nj[4]), Nj[5])
    f3 = jnp.exp(Nj[3] - m3)
    f4 = jnp.exp(Nj[4] - m3)
    f5 = jnp.exp(Nj[5] - m3)
    nn_ref[...] = (Nj[0] * f3 + Nj[1] * f4 + Nj[2] * f5) / (f3 + f4 + f5)


@functools.lru_cache(maxsize=None)
def _make_tc(n_pad: int):
    grid = (n_pad // T_,)
    TM = T_ * M_
    KA = K_ * A_
    KB = K_ * B_

    def vmem(shape):
        return pl.BlockSpec(shape, lambda *_: (0,) * len(shape))

    smem = pl.BlockSpec(memory_space=pltpu.SMEM)
    in_specs = [
        pl.BlockSpec((TM, A_), lambda i: (i, 0)),
        pl.BlockSpec((TM, B_), lambda i: (i, 0)),
        pl.BlockSpec((T_, A_), lambda i: (i, 0)),
        vmem((A_, KA)), vmem((A_, KA)), vmem((A_, KB)),
        vmem((B_, KA)), vmem((B_, KA)), vmem((B_, KB)),
        vmem((A_, KA)), vmem((A_, KA)), vmem((A_, KB)),
        vmem((8, KA)), vmem((8, KA)), vmem((8, KB)),
        vmem((8, KA)), vmem((8, KA)),
        smem, smem, smem, smem,
    ]
    out_specs = (
        pl.BlockSpec((T_, A_), lambda i: (i, 0)),
        pl.BlockSpec((T_, M_, B_), lambda i: (i, 0, 0)),
    )
    out_shape = (
        jax.ShapeDtypeStruct((n_pad, A_), jnp.float32),
        jax.ShapeDtypeStruct((n_pad, M_, B_), jnp.float32),
    )
    return pl.pallas_call(
        _tc_body,
        grid=grid,
        in_specs=in_specs,
        out_specs=out_specs,
        out_shape=out_shape,
    )


def _row8(v):
    return jnp.broadcast_to(v[None, :], (8, v.shape[0]))


def kernel(atom_in_fea, nbr_fea, nbr_fea_idx, params):
    N = atom_in_fea.shape[0]
    n_pad = ((N + 255) // 256) * 256

    W = params["fc_W"]
    s1 = params["bn1_g"] * lax.rsqrt(params["bn1_rv"] + 1e-5)
    Wf = W.transpose(0, 2, 1) * s1[:, None, :]
    bf = (params["fc_b"] - params["bn1_rm"]) * s1 + params["bn1_b"]

    WF = jnp.concatenate([Wf[k, :, :A_] for k in range(K_)], axis=1)
    WC = jnp.concatenate([Wf[k, :, A_:2 * A_] for k in range(K_)], axis=1)
    WE = jnp.concatenate([Wf[k, :, 2 * A_:] for k in range(K_)], axis=1)
    bF = jnp.concatenate([bf[k, :A_] for k in range(K_)])
    bC = jnp.concatenate([bf[k, A_:2 * A_] for k in range(K_)])
    bE = jnp.concatenate([bf[k, 2 * A_:] for k in range(K_)])

    wsF, wnF, weF = WF[:A_], WF[A_:2 * A_], WF[2 * A_:]
    wsC, wnC, weC = WC[:A_], WC[A_:2 * A_], WC[2 * A_:]
    wsE, wnE, weE = WE[:A_], WE[A_:2 * A_], WE[2 * A_:]

    s2k = params["bn2_g"] * lax.rsqrt(params["bn2_rv"] + 1e-5)
    t2k = params["bn2_b"] - params["bn2_rm"] * s2k
    s2 = jnp.concatenate([s2k[k] for k in range(K_)])
    t2 = jnp.concatenate([t2k[k] for k in range(K_)])

    atom_p = jnp.pad(atom_in_fea, ((0, n_pad - N), (0, 0)))
    idx_p = jnp.pad(nbr_fea_idx, ((0, n_pad - N), (0, 0))).reshape(-1)
    nbr_p = jnp.pad(nbr_fea, ((0, n_pad - N), (0, 0), (0, 0)))
    nbr2d = nbr_p.reshape(n_pad * M_, B_)

    gathered = _make_gather(n_pad)(atom_p, idx_p)

    out_p, nn_p = _make_tc(n_pad)(
        gathered, nbr2d, atom_p,
        wnF, wnC, wnE, weF, weC, weE, wsF, wsC, wsE,
        _row8(bF), _row8(bC), _row8(bE), _row8(s2), _row8(t2),
        params["atom_fc_W"], params["atom_fc_b"],
        params["nbr_fc_W"], params["nbr_fc_b"],
    )
    return out_p[:N], nn_p[:N]
